# baseline (speedup 1.0000x reference)
import functools

import jax
import jax.numpy as jnp
import numpy as np
from jax import lax
from jax.experimental import pallas as pl
from jax.experimental.pallas import tpu as pltpu
from jax.experimental.pallas import tpu_sc as plsc

B = 4096
D_DENSE = 13
N_SPARSE = 26
VOCAB = 100000
EMB = 64
N = N_SPARSE + 1
PROJ = 128

NW = 32
ROWS_PER_W = (N_SPARSE * B) // NW
CHUNK = 128
NCHUNK = ROWS_PER_W // CHUNK

BB = 512


def _sc_gather(gidx2, tables_pair):
    mesh = plsc.VectorSubcoreMesh(core_axis_name="c", subcore_axis_name="s")

    @functools.partial(
        pl.kernel,
        mesh=mesh,
        out_type=jax.ShapeDtypeStruct((N_SPARSE * B, 2 * EMB), jnp.float32),
        scratch_types=[
            pltpu.VMEM((NCHUNK, CHUNK), jnp.int32),
            pltpu.VMEM((2, CHUNK, 2 * EMB), jnp.float32),
            pltpu.SemaphoreType.DMA,
            pltpu.SemaphoreType.DMA,
            pltpu.SemaphoreType.DMA,
            pltpu.SemaphoreType.DMA,
        ],
    )
    def gather_kernel(gidx_hbm, table_hbm, out_hbm, idx_v, rows_v,
                      g0, g1, o0, o1):
        wid = lax.axis_index("s") * 2 + lax.axis_index("c")
        pltpu.sync_copy(gidx_hbm.at[wid], idx_v)
        gsem = (g0, g1)
        osem = (o0, o1)

        gh = [None, None]
        oh = [None, None]
        gh[0] = pltpu.async_copy(table_hbm.at[idx_v.at[0]], rows_v.at[0],
                                 gsem[0])
        for j in range(NCHUNK):
            b = j & 1
            nb = 1 - b
            if j + 1 < NCHUNK:
                if oh[nb] is not None:
                    oh[nb].wait()
                gh[nb] = pltpu.async_copy(
                    table_hbm.at[idx_v.at[j + 1]], rows_v.at[nb], gsem[nb])
            gh[b].wait()
            oh[b] = pltpu.async_copy(
                rows_v.at[b],
                out_hbm.at[pl.ds(wid * ROWS_PER_W + j * CHUNK, CHUNK)],
                osem[b])
        oh[0].wait()
        oh[1].wait()

    return gather_kernel(gidx2, tables_pair)


def _tc_body(dense_ref, emb_ref, par_ref, W0_ref, b0_ref, W1_ref, b1_ref,
             W2_ref, b2_ref, Wph_ref, S2_ref, bp_ref, out_ref):
    f32 = jnp.float32
    h = jnp.maximum(
        jnp.dot(dense_ref[...], W0_ref[...], preferred_element_type=f32)
        + b0_ref[...], 0.0)
    h = jnp.maximum(
        jnp.dot(h, W1_ref[...], preferred_element_type=f32) + b1_ref[...], 0.0)
    h = jnp.dot(h, W2_ref[...], preferred_element_type=f32) + b2_ref[...]
    sel = par_ref[...] == 1
    embn = emb_ref[...].reshape(BB, N_SPARSE, 2 * EMB)
    emb = jnp.where(sel, embn[:, :, EMB:], embn[:, :, :EMB])
    T3 = jnp.concatenate([h[:, None, :], emb], axis=1)
    Z = lax.dot_general(T3, T3, (((2,), (2,)), ((0,), (0,))),
                        preferred_element_type=f32)
    Zr = Z.reshape(BB, N * N)
    out = (jnp.dot(h, Wph_ref[...], preferred_element_type=f32)
           + jnp.dot(Zr, S2_ref[...], preferred_element_type=f32)
           + bp_ref[...])
    out_ref[...] = out


def _tc_dense(dense_p, emb3, par2, W0p, b0, W1, b1, W2, b2, Wph, S2, bp):
    grid = (B // BB,)
    full = lambda shape: pl.BlockSpec(shape, lambda i: (0,) * len(shape))
    return pl.pallas_call(
        _tc_body,
        grid=grid,
        in_specs=[
            pl.BlockSpec((BB, 16), lambda i: (i, 0)),
            pl.BlockSpec((BB * N_SPARSE, 2 * EMB), lambda i: (i, 0)),
            pl.BlockSpec((BB, N_SPARSE, 1), lambda i: (i, 0, 0)),
            full((16, 512)),
            full((1, 512)),
            full((512, 256)),
            full((1, 256)),
            full((256, EMB)),
            full((1, EMB)),
            full((EMB, PROJ)),
            full((N * N, PROJ)),
            full((1, PROJ)),
        ],
        out_specs=pl.BlockSpec((BB, PROJ), lambda i: (i, 0)),
        out_shape=jax.ShapeDtypeStruct((B, PROJ), jnp.float32),
    )(dense_p, emb3, par2, W0p, b0, W1, b1, W2, b2, Wph, S2, bp)


_TRIU_ROW, _TRIU_COL = np.triu_indices(N, k=1)
_TRIU_FLAT = np.asarray(_TRIU_ROW * N + _TRIU_COL, dtype=np.int32)


def kernel(dense, emb_indices, tables, W0, b0, W1, b1, W2, b2, Wp, bp):
    offs = (jnp.arange(N_SPARSE, dtype=jnp.int32) * VOCAB)[:, None]
    gflat = (emb_indices.astype(jnp.int32) + offs).T
    gidx = (gflat >> 1).reshape(NW, NCHUNK, CHUNK)
    par2 = (gflat & 1).reshape(B, N_SPARSE, 1)
    tables_pair = tables.reshape(N_SPARSE * VOCAB // 2, 2 * EMB)
    dense_p = jnp.pad(dense, ((0, 0), (0, 16 - D_DENSE)))
    W0p = jnp.pad(W0, ((0, 16 - D_DENSE), (0, 0)))
    S2 = jnp.zeros((N * N, PROJ), jnp.float32).at[_TRIU_FLAT].set(Wp[EMB:])
    Wph = Wp[:EMB]

    emb_flat = _sc_gather(gidx, tables_pair)

    return _tc_dense(dense_p, emb_flat, par2, W0p, b0.reshape(1, -1), W1,
                     b1.reshape(1, -1), W2, b2.reshape(1, -1), Wph, S2,
                     bp.reshape(1, -1))


# speedup vs baseline: 1.0202x; 1.0202x over previous
import functools

import jax
import jax.numpy as jnp
import numpy as np
from jax import lax
from jax.experimental import pallas as pl
from jax.experimental.pallas import tpu as pltpu
from jax.experimental.pallas import tpu_sc as plsc

B = 4096
D_DENSE = 13
N_SPARSE = 26
VOCAB = 100000
EMB = 64
N = N_SPARSE + 1
PROJ = 128

NW = 32
ROWS_PER_W = (N_SPARSE * B) // NW
CHUNK = 128
NCHUNK = ROWS_PER_W // CHUNK

BB = 512


def _sc_gather(gidx2, tables_pair):
    mesh = plsc.VectorSubcoreMesh(core_axis_name="c", subcore_axis_name="s")

    @functools.partial(
        pl.kernel,
        mesh=mesh,
        out_type=jax.ShapeDtypeStruct((N_SPARSE * B, EMB), jnp.float32),
        compiler_params=pltpu.CompilerParams(use_tc_tiling_on_sc=False),
        scratch_types=[
            pltpu.VMEM((NCHUNK, CHUNK), jnp.int32),
            pltpu.VMEM((2, CHUNK, EMB), jnp.float32),
            pltpu.SemaphoreType.DMA,
            pltpu.SemaphoreType.DMA,
            pltpu.SemaphoreType.DMA,
            pltpu.SemaphoreType.DMA,
        ],
    )
    def gather_kernel(gidx_hbm, table_hbm, out_hbm, idx_v, rows_v,
                      g0, g1, o0, o1):
        wid = lax.axis_index("s") * 2 + lax.axis_index("c")
        pltpu.sync_copy(gidx_hbm.at[wid], idx_v)
        gsem = (g0, g1)
        osem = (o0, o1)

        gh = [None, None]
        oh = [None, None]
        gh[0] = pltpu.async_copy(table_hbm.at[idx_v.at[0]], rows_v.at[0],
                                 gsem[0])
        for j in range(NCHUNK):
            b = j & 1
            nb = 1 - b
            if j + 1 < NCHUNK:
                if oh[nb] is not None:
                    oh[nb].wait()
                gh[nb] = pltpu.async_copy(
                    table_hbm.at[idx_v.at[j + 1]], rows_v.at[nb], gsem[nb])
            gh[b].wait()
            oh[b] = pltpu.async_copy(
                rows_v.at[b],
                out_hbm.at[pl.ds(wid * ROWS_PER_W + j * CHUNK, CHUNK)],
                osem[b])
        oh[0].wait()
        oh[1].wait()

    return gather_kernel(gidx2, tables_pair)


def _tc_body(dense_ref, emb_ref, W0_ref, b0_ref, W1_ref, b1_ref,
             W2_ref, b2_ref, Wph_ref, S2_ref, bp_ref, out_ref):
    f32 = jnp.float32
    h = jnp.maximum(
        jnp.dot(dense_ref[...], W0_ref[...], preferred_element_type=f32)
        + b0_ref[...], 0.0)
    h = jnp.maximum(
        jnp.dot(h, W1_ref[...], preferred_element_type=f32) + b1_ref[...], 0.0)
    h = jnp.dot(h, W2_ref[...], preferred_element_type=f32) + b2_ref[...]
    emb = emb_ref[...].reshape(BB, N_SPARSE, EMB)
    T3 = jnp.concatenate([h[:, None, :], emb], axis=1)
    Z = lax.dot_general(T3, T3, (((2,), (2,)), ((0,), (0,))),
                        preferred_element_type=f32)
    Zr = Z.reshape(BB, N * N)
    out = (jnp.dot(h, Wph_ref[...], preferred_element_type=f32)
           + jnp.dot(Zr, S2_ref[...], preferred_element_type=f32)
           + bp_ref[...])
    out_ref[...] = out


def _tc_dense(dense_p, emb3, W0p, b0, W1, b1, W2, b2, Wph, S2, bp):
    grid = (B // BB,)
    full = lambda shape: pl.BlockSpec(shape, lambda i: (0,) * len(shape))
    return pl.pallas_call(
        _tc_body,
        grid=grid,
        in_specs=[
            pl.BlockSpec((BB, 16), lambda i: (i, 0)),
            pl.BlockSpec((BB * N_SPARSE, EMB), lambda i: (i, 0)),
            full((16, 512)),
            full((1, 512)),
            full((512, 256)),
            full((1, 256)),
            full((256, EMB)),
            full((1, EMB)),
            full((EMB, PROJ)),
            full((N * N, PROJ)),
            full((1, PROJ)),
        ],
        out_specs=pl.BlockSpec((BB, PROJ), lambda i: (i, 0)),
        out_shape=jax.ShapeDtypeStruct((B, PROJ), jnp.float32),
    )(dense_p, emb3, W0p, b0, W1, b1, W2, b2, Wph, S2, bp)


_TRIU_ROW, _TRIU_COL = np.triu_indices(N, k=1)
_TRIU_FLAT = np.asarray(_TRIU_ROW * N + _TRIU_COL, dtype=np.int32)


def kernel(dense, emb_indices, tables, W0, b0, W1, b1, W2, b2, Wp, bp):
    offs = (jnp.arange(N_SPARSE, dtype=jnp.int32) * VOCAB)[:, None]
    gflat = (emb_indices.astype(jnp.int32) + offs).T
    gidx = gflat.reshape(NW, NCHUNK, CHUNK)
    tables_pair = tables.reshape(N_SPARSE * VOCAB, EMB)
    dense_p = jnp.pad(dense, ((0, 0), (0, 16 - D_DENSE)))
    W0p = jnp.pad(W0, ((0, 16 - D_DENSE), (0, 0)))
    S2 = jnp.zeros((N * N, PROJ), jnp.float32).at[_TRIU_FLAT].set(Wp[EMB:])
    Wph = Wp[:EMB]

    emb_flat = _sc_gather(gidx, tables_pair)

    return _tc_dense(dense_p, emb_flat, W0p, b0.reshape(1, -1), W1,
                     b1.reshape(1, -1), W2, b2.reshape(1, -1), Wph, S2,
                     bp.reshape(1, -1))
